# Initial kernel scaffold; baseline (speedup 1.0000x reference)
import functools

import jax
import jax.numpy as jnp
from jax import lax
from jax.experimental import pallas as pl
from jax.experimental.pallas import tpu as pltpu
from jax.experimental.pallas import tpu_sc as plsc

N_NODES = 10000
N_EDGES = 320000
D = 128

_NC = 2
_NS = 16
_NW = _NC * _NS
_CHUNK = 128
_NCH = N_EDGES // _CHUNK
_CPW = -(-_NCH // _NW)
_ROWS_PER_TILE = N_NODES // _NS

_sc_mesh = plsc.VectorSubcoreMesh(core_axis_name="c", subcore_axis_name="s")


def _pq_body(x_ref, wa_ref, wb_ref, be1_ref, p_ref, q_ref):
    x = x_ref[...]
    half_b = 0.5 * be1_ref[...]
    p_ref[...] = jnp.dot(x, wa_ref[...], preferred_element_type=jnp.float32) + half_b
    q_ref[...] = jnp.dot(x, wb_ref[...], preferred_element_type=jnp.float32) + half_b


def _node_proj(x, wa, wb, be1):
    nb = 2000
    grid = N_NODES // nb
    return pl.pallas_call(
        _pq_body,
        grid=(grid,),
        in_specs=[
            pl.BlockSpec((nb, D), lambda i: (i, 0)),
            pl.BlockSpec((D, D), lambda i: (0, 0)),
            pl.BlockSpec((D, D), lambda i: (0, 0)),
            pl.BlockSpec((1, D), lambda i: (0, 0)),
        ],
        out_specs=[
            pl.BlockSpec((nb, D), lambda i: (i, 0)),
            pl.BlockSpec((nb, D), lambda i: (i, 0)),
        ],
        out_shape=[
            jax.ShapeDtypeStruct((N_NODES, D), jnp.float32),
            jax.ShapeDtypeStruct((N_NODES, D), jnp.float32),
        ],
    )(x, wa, wb, be1.reshape(1, D))


def _gather_body(p_hbm, q_hbm, src_hbm, dst_hbm, z_hbm, idx_s, idx_d, rows, sem):
    wid = lax.axis_index("s") * _NC + lax.axis_index("c")

    def chunk(i, carry):
        base = (wid * _CPW + i) * _CHUNK

        @pl.when(base < N_EDGES)
        def _():
            pltpu.sync_copy(src_hbm.at[pl.ds(base, _CHUNK)], idx_s)
            pltpu.sync_copy(dst_hbm.at[pl.ds(base, _CHUNK)], idx_d)
            pltpu.async_copy(p_hbm.at[idx_s], rows, sem).wait()
            pltpu.async_copy(q_hbm.at[idx_d], rows, sem, add=True).wait()
            pltpu.sync_copy(rows, z_hbm.at[pl.ds(base, _CHUNK)])

        return carry

    lax.fori_loop(0, _CPW, chunk, 0)


_gather = functools.partial(
    pl.kernel,
    out_type=jax.ShapeDtypeStruct((N_EDGES, D), jnp.float32),
    mesh=_sc_mesh,
    scratch_types=[
        pltpu.VMEM((_CHUNK,), jnp.int32),
        pltpu.VMEM((_CHUNK,), jnp.int32),
        pltpu.VMEM((_CHUNK, D), jnp.float32),
        pltpu.SemaphoreType.DMA,
    ],
)(_gather_body)


def _edge_mlp_body(z_ref, ea_ref, wc_ref, we2_ref, be2_ref, out_ref):
    z = z_ref[...] + jnp.dot(
        ea_ref[...], wc_ref[...], preferred_element_type=jnp.float32
    )
    h = z * jax.nn.sigmoid(z)
    out_ref[...] = (
        jnp.dot(h, we2_ref[...], preferred_element_type=jnp.float32) + be2_ref[...]
    )


def _edge_mlp(z, edge_attr, wc, we2, be2):
    eb = 2000
    grid = N_EDGES // eb
    return pl.pallas_call(
        _edge_mlp_body,
        grid=(grid,),
        in_specs=[
            pl.BlockSpec((eb, D), lambda i: (i, 0)),
            pl.BlockSpec((eb, 4), lambda i: (i, 0)),
            pl.BlockSpec((4, D), lambda i: (0, 0)),
            pl.BlockSpec((D, D), lambda i: (0, 0)),
            pl.BlockSpec((1, D), lambda i: (0, 0)),
        ],
        out_specs=pl.BlockSpec((eb, D), lambda i: (i, 0)),
        out_shape=jax.ShapeDtypeStruct((N_EDGES, D), jnp.float32),
    )(z, edge_attr, wc, we2, be2.reshape(1, D))


def _scatter_body(e2_hbm, dst_hbm, zero_hbm, out_hbm, idx_d, rows, acc, sem):
    cid = lax.axis_index("c")
    sid = lax.axis_index("s")
    wid = sid * _NC + cid
    row0 = sid * _ROWS_PER_TILE

    pltpu.sync_copy(
        zero_hbm.at[pl.ds(row0, _ROWS_PER_TILE)],
        acc.at[pl.ds(row0, _ROWS_PER_TILE)],
    )
    plsc.subcore_barrier()

    def chunk(i, carry):
        base = (wid * _CPW + i) * _CHUNK

        @pl.when(base < N_EDGES)
        def _():
            pltpu.sync_copy(dst_hbm.at[pl.ds(base, _CHUNK)], idx_d)
            pltpu.sync_copy(e2_hbm.at[pl.ds(base, _CHUNK)], rows)
            pltpu.sync_copy(rows, acc.at[idx_d], add=True)

        return carry

    lax.fori_loop(0, _CPW, chunk, 0)
    plsc.subcore_barrier()
    pltpu.sync_copy(
        acc.at[pl.ds(row0, _ROWS_PER_TILE)],
        out_hbm.at[cid, pl.ds(row0, _ROWS_PER_TILE)],
    )


_scatter = functools.partial(
    pl.kernel,
    out_type=jax.ShapeDtypeStruct((_NC, N_NODES, D), jnp.float32),
    mesh=_sc_mesh,
    scratch_types=[
        pltpu.VMEM((_CHUNK,), jnp.int32),
        pltpu.VMEM((_CHUNK, D), jnp.float32),
        pltpu.VMEM_SHARED((N_NODES, D), jnp.float32),
        pltpu.SemaphoreType.DMA,
    ],
)(_scatter_body)


def _node_mlp_body(
    x_ref, p0_ref, p1_ref, wn1a_ref, wn1b_ref, bn1_ref, wn2_ref, bn2_ref, out_ref
):
    x = x_ref[...]
    msg = p0_ref[...] + p1_ref[...]
    h = (
        jnp.dot(x, wn1a_ref[...], preferred_element_type=jnp.float32)
        + jnp.dot(msg, wn1b_ref[...], preferred_element_type=jnp.float32)
        + bn1_ref[...]
    )
    h = h * jax.nn.sigmoid(h)
    out_ref[...] = (
        x + jnp.dot(h, wn2_ref[...], preferred_element_type=jnp.float32) + bn2_ref[...]
    )


def _node_mlp(x, partials, wn1a, wn1b, bn1, wn2, bn2):
    nb = 2000
    grid = N_NODES // nb
    return pl.pallas_call(
        _node_mlp_body,
        grid=(grid,),
        in_specs=[
            pl.BlockSpec((nb, D), lambda i: (i, 0)),
            pl.BlockSpec((nb, D), lambda i: (i, 0)),
            pl.BlockSpec((nb, D), lambda i: (i, 0)),
            pl.BlockSpec((D, D), lambda i: (0, 0)),
            pl.BlockSpec((D, D), lambda i: (0, 0)),
            pl.BlockSpec((1, D), lambda i: (0, 0)),
            pl.BlockSpec((D, D), lambda i: (0, 0)),
            pl.BlockSpec((1, D), lambda i: (0, 0)),
        ],
        out_specs=pl.BlockSpec((nb, D), lambda i: (i, 0)),
        out_shape=jax.ShapeDtypeStruct((N_NODES, D), jnp.float32),
    )(x, partials[0], partials[1], wn1a, wn1b, bn1.reshape(1, D), wn2, bn2.reshape(1, D))


@jax.jit
def kernel(x, edge_index, edge_attr, We1, be1, We2, be2, Wn1, bn1, Wn2, bn2):
    src = edge_index[0].astype(jnp.int32)
    dst = edge_index[1].astype(jnp.int32)
    wa = We1[:D]
    wb = We1[D : 2 * D]
    wc = We1[2 * D :]
    wn1a = Wn1[:D]
    wn1b = Wn1[D:]
    zeros = jnp.zeros((N_NODES, D), jnp.float32)

    p, q = _node_proj(x, wa, wb, be1)
    z = _gather(p, q, src, dst)
    e2 = _edge_mlp(z, edge_attr, wc, We2, be2)
    partials = _scatter(e2, dst, zeros)
    return _node_mlp(x, partials, wn1a, wn1b, bn1, Wn2, bn2)


# speedup vs baseline: 3.5018x; 3.5018x over previous
import functools

import jax
import jax.numpy as jnp
from jax import lax
from jax.experimental import pallas as pl
from jax.experimental.pallas import tpu as pltpu
from jax.experimental.pallas import tpu_sc as plsc

N_NODES = 10000
N_EDGES = 320000
D = 128

_NC = 2
_NS = 16
_NW = _NC * _NS
_CHUNK = 128
_NCH = N_EDGES // _CHUNK
_CPW = -(-_NCH // _NW)
_N_PAD = 10240
_ROWS_PER_TILE = _N_PAD // _NS

_sc_mesh = plsc.VectorSubcoreMesh(core_axis_name="c", subcore_axis_name="s")


def _pq_body(x_ref, wa_ref, wb_ref, be1_ref, p_ref, q_ref):
    x = x_ref[...]
    half_b = 0.5 * be1_ref[...]
    p_ref[...] = jnp.dot(x, wa_ref[...], preferred_element_type=jnp.float32) + half_b
    q_ref[...] = jnp.dot(x, wb_ref[...], preferred_element_type=jnp.float32) + half_b


def _node_proj(x, wa, wb, be1):
    nb = 2000
    grid = N_NODES // nb
    return pl.pallas_call(
        _pq_body,
        grid=(grid,),
        in_specs=[
            pl.BlockSpec((nb, D), lambda i: (i, 0)),
            pl.BlockSpec((D, D), lambda i: (0, 0)),
            pl.BlockSpec((D, D), lambda i: (0, 0)),
            pl.BlockSpec((1, D), lambda i: (0, 0)),
        ],
        out_specs=[
            pl.BlockSpec((nb, D), lambda i: (i, 0)),
            pl.BlockSpec((nb, D), lambda i: (i, 0)),
        ],
        out_shape=[
            jax.ShapeDtypeStruct((N_NODES, D), jnp.float32),
            jax.ShapeDtypeStruct((N_NODES, D), jnp.float32),
        ],
    )(x, wa, wb, be1.reshape(1, D))


def _gather_body(p_hbm, q_hbm, src_hbm, dst_hbm, z_hbm, idx_s, idx_d, rows, sem):
    wid = lax.axis_index("s") * _NC + lax.axis_index("c")

    def chunk(i, carry):
        base = (wid * _CPW + i) * _CHUNK

        @pl.when(base < N_EDGES)
        def _():
            pltpu.sync_copy(src_hbm.at[pl.ds(base, _CHUNK)], idx_s)
            pltpu.sync_copy(dst_hbm.at[pl.ds(base, _CHUNK)], idx_d)
            pltpu.async_copy(p_hbm.at[idx_s], rows, sem).wait()
            pltpu.async_copy(q_hbm.at[idx_d], rows, sem, add=True).wait()
            pltpu.sync_copy(rows, z_hbm.at[pl.ds(base, _CHUNK)])

        return carry

    lax.fori_loop(0, _CPW, chunk, 0)


_gather = functools.partial(
    pl.kernel,
    out_type=jax.ShapeDtypeStruct((N_EDGES, D), jnp.float32),
    mesh=_sc_mesh,
    scratch_types=[
        pltpu.VMEM((_CHUNK,), jnp.int32),
        pltpu.VMEM((_CHUNK,), jnp.int32),
        pltpu.VMEM((_CHUNK, D), jnp.float32),
        pltpu.SemaphoreType.DMA,
    ],
)(_gather_body)


def _edge_mlp_body(z_ref, ea_ref, wc_ref, we2_ref, be2_ref, out_ref):
    z = z_ref[...] + jnp.dot(
        ea_ref[...], wc_ref[...], preferred_element_type=jnp.float32
    )
    h = z * jax.nn.sigmoid(z)
    out_ref[...] = (
        jnp.dot(h, we2_ref[...], preferred_element_type=jnp.float32) + be2_ref[...]
    )


def _edge_mlp(z, edge_attr, wc, we2, be2):
    eb = 2000
    grid = N_EDGES // eb
    return pl.pallas_call(
        _edge_mlp_body,
        grid=(grid,),
        in_specs=[
            pl.BlockSpec((eb, D), lambda i: (i, 0)),
            pl.BlockSpec((eb, 4), lambda i: (i, 0)),
            pl.BlockSpec((4, D), lambda i: (0, 0)),
            pl.BlockSpec((D, D), lambda i: (0, 0)),
            pl.BlockSpec((1, D), lambda i: (0, 0)),
        ],
        out_specs=pl.BlockSpec((eb, D), lambda i: (i, 0)),
        out_shape=jax.ShapeDtypeStruct((N_EDGES, D), jnp.float32),
    )(z, edge_attr, wc, we2, be2.reshape(1, D))


def _scatter_body(e2_hbm, dst_hbm, zero_hbm, out_hbm, idx_d, rows, acc, sem):
    cid = lax.axis_index("c")
    sid = lax.axis_index("s")
    wid = sid * _NC + cid
    row0 = sid * _ROWS_PER_TILE

    pltpu.sync_copy(
        zero_hbm.at[pl.ds(row0, _ROWS_PER_TILE)],
        acc.at[pl.ds(row0, _ROWS_PER_TILE)],
    )
    plsc.subcore_barrier()

    def chunk(i, carry):
        base = (wid * _CPW + i) * _CHUNK

        @pl.when(base < N_EDGES)
        def _():
            pltpu.sync_copy(dst_hbm.at[pl.ds(base, _CHUNK)], idx_d)
            pltpu.sync_copy(e2_hbm.at[pl.ds(base, _CHUNK)], rows)
            pltpu.sync_copy(rows, acc.at[idx_d], add=True)

        return carry

    lax.fori_loop(0, _CPW, chunk, 0)
    plsc.subcore_barrier()
    pltpu.sync_copy(
        acc.at[pl.ds(row0, _ROWS_PER_TILE)],
        out_hbm.at[cid, pl.ds(row0, _ROWS_PER_TILE)],
    )


_scatter = functools.partial(
    pl.kernel,
    out_type=jax.ShapeDtypeStruct((_NC, _N_PAD, D), jnp.float32),
    mesh=_sc_mesh,
    scratch_types=[
        pltpu.VMEM((_CHUNK,), jnp.int32),
        pltpu.VMEM((_CHUNK, D), jnp.float32),
        pltpu.VMEM_SHARED((_N_PAD, D), jnp.float32),
        pltpu.SemaphoreType.DMA,
    ],
)(_scatter_body)


def _node_mlp_body(
    x_ref, p0_ref, p1_ref, wn1a_ref, wn1b_ref, bn1_ref, wn2_ref, bn2_ref, out_ref
):
    x = x_ref[...]
    msg = p0_ref[...] + p1_ref[...]
    h = (
        jnp.dot(x, wn1a_ref[...], preferred_element_type=jnp.float32)
        + jnp.dot(msg, wn1b_ref[...], preferred_element_type=jnp.float32)
        + bn1_ref[...]
    )
    h = h * jax.nn.sigmoid(h)
    out_ref[...] = (
        x + jnp.dot(h, wn2_ref[...], preferred_element_type=jnp.float32) + bn2_ref[...]
    )


def _node_mlp(x, partials, wn1a, wn1b, bn1, wn2, bn2):
    nb = 2000
    grid = N_NODES // nb
    return pl.pallas_call(
        _node_mlp_body,
        grid=(grid,),
        in_specs=[
            pl.BlockSpec((nb, D), lambda i: (i, 0)),
            pl.BlockSpec((nb, D), lambda i: (i, 0)),
            pl.BlockSpec((nb, D), lambda i: (i, 0)),
            pl.BlockSpec((D, D), lambda i: (0, 0)),
            pl.BlockSpec((D, D), lambda i: (0, 0)),
            pl.BlockSpec((1, D), lambda i: (0, 0)),
            pl.BlockSpec((D, D), lambda i: (0, 0)),
            pl.BlockSpec((1, D), lambda i: (0, 0)),
        ],
        out_specs=pl.BlockSpec((nb, D), lambda i: (i, 0)),
        out_shape=jax.ShapeDtypeStruct((N_NODES, D), jnp.float32),
    )(x, partials[0], partials[1], wn1a, wn1b, bn1.reshape(1, D), wn2, bn2.reshape(1, D))


@jax.jit
def kernel(x, edge_index, edge_attr, We1, be1, We2, be2, Wn1, bn1, Wn2, bn2):
    src = edge_index[0].astype(jnp.int32)
    dst = edge_index[1].astype(jnp.int32)
    wa = We1[:D]
    wb = We1[D : 2 * D]
    wc = We1[2 * D :]
    wn1a = Wn1[:D]
    wn1b = Wn1[D:]
    zeros = jnp.zeros((_N_PAD, D), jnp.float32)

    p, q = _node_proj(x, wa, wb, be1)
    z = _gather(p, q, src, dst)
    e2 = _edge_mlp(z, edge_attr, wc, We2, be2)
    partials = _scatter(e2, dst, zeros)
    partials = partials[:, :N_NODES]
    return _node_mlp(x, partials, wn1a, wn1b, bn1, Wn2, bn2)


# speedup vs baseline: 4.4218x; 1.2627x over previous
import functools

import jax
import jax.numpy as jnp
from jax import lax
from jax.experimental import pallas as pl
from jax.experimental.pallas import tpu as pltpu
from jax.experimental.pallas import tpu_sc as plsc

N_NODES = 10000
N_EDGES = 320000
D = 128

_NC = 2
_NS = 16
_NW = _NC * _NS
_CHUNK = 128
_NCH = N_EDGES // _CHUNK
_K = 4
_CPW = 80
_E_PAD = _NW * _CPW * _CHUNK
_N_PAD = 10240
_ROWS_PER_TILE = _N_PAD // _NS

_sc_mesh = plsc.VectorSubcoreMesh(core_axis_name="c", subcore_axis_name="s")


def _pq_body(x_ref, wa_ref, wb_ref, be1_ref, p_ref, q_ref):
    x = x_ref[...]
    half_b = 0.5 * be1_ref[...]
    p_ref[...] = jnp.dot(x, wa_ref[...], preferred_element_type=jnp.float32) + half_b
    q_ref[...] = jnp.dot(x, wb_ref[...], preferred_element_type=jnp.float32) + half_b


def _node_proj(x, wa, wb, be1):
    nb = 2000
    grid = N_NODES // nb
    return pl.pallas_call(
        _pq_body,
        grid=(grid,),
        in_specs=[
            pl.BlockSpec((nb, D), lambda i: (i, 0)),
            pl.BlockSpec((D, D), lambda i: (0, 0)),
            pl.BlockSpec((D, D), lambda i: (0, 0)),
            pl.BlockSpec((1, D), lambda i: (0, 0)),
        ],
        out_specs=[
            pl.BlockSpec((nb, D), lambda i: (i, 0)),
            pl.BlockSpec((nb, D), lambda i: (i, 0)),
        ],
        out_shape=[
            jax.ShapeDtypeStruct((N_NODES, D), jnp.float32),
            jax.ShapeDtypeStruct((N_NODES, D), jnp.float32),
        ],
    )(x, wa, wb, be1.reshape(1, D))


def _gather_body(p_hbm, q_hbm, src2_hbm, dst2_hbm, z_hbm, isrc, idst, rows, sem, semw):
    wid = lax.axis_index("s") * _NC + lax.axis_index("c")
    c0 = wid * _CPW
    pltpu.sync_copy(src2_hbm.at[pl.ds(c0, _CPW)], isrc)
    pltpu.sync_copy(dst2_hbm.at[pl.ds(c0, _CPW)], idst)

    def group(g, carry):
        def ok(b):
            return (c0 + g * _K + b) * _CHUNK < N_EDGES

        for b in range(_K):
            @pl.when(ok(b))
            def _(b=b):
                pltpu.async_copy(p_hbm.at[isrc.at[g * _K + b]], rows.at[b], sem)
        for b in range(_K):
            @pl.when(ok(b))
            def _(b=b):
                pltpu.make_async_copy(
                    p_hbm.at[isrc.at[g * _K + b]], rows.at[b], sem
                ).wait()
        for b in range(_K):
            @pl.when(ok(b))
            def _(b=b):
                pltpu.async_copy(
                    q_hbm.at[idst.at[g * _K + b]], rows.at[b], sem, add=True
                )
        for b in range(_K):
            @pl.when(ok(b))
            def _(b=b):
                pltpu.make_async_copy(
                    q_hbm.at[idst.at[g * _K + b]], rows.at[b], sem
                ).wait()
        for b in range(_K):
            @pl.when(ok(b))
            def _(b=b):
                base = (c0 + g * _K + b) * _CHUNK
                pltpu.async_copy(rows.at[b], z_hbm.at[pl.ds(base, _CHUNK)], semw)
        for b in range(_K):
            @pl.when(ok(b))
            def _(b=b):
                base = (c0 + g * _K + b) * _CHUNK
                pltpu.make_async_copy(
                    rows.at[b], z_hbm.at[pl.ds(base, _CHUNK)], semw
                ).wait()
        return carry

    lax.fori_loop(0, _CPW // _K, group, 0)


_gather = functools.partial(
    pl.kernel,
    out_type=jax.ShapeDtypeStruct((N_EDGES, D), jnp.float32),
    mesh=_sc_mesh,
    scratch_types=[
        pltpu.VMEM((_CPW, _CHUNK), jnp.int32),
        pltpu.VMEM((_CPW, _CHUNK), jnp.int32),
        pltpu.VMEM((_K, _CHUNK, D), jnp.float32),
        pltpu.SemaphoreType.DMA,
        pltpu.SemaphoreType.DMA,
    ],
)(_gather_body)


def _edge_mlp_body(z_ref, ea_ref, wc_ref, we2_ref, be2_ref, out_ref):
    z = z_ref[...] + jnp.dot(
        ea_ref[...], wc_ref[...], preferred_element_type=jnp.float32
    )
    h = z * jax.nn.sigmoid(z)
    out_ref[...] = (
        jnp.dot(h, we2_ref[...], preferred_element_type=jnp.float32) + be2_ref[...]
    )


def _edge_mlp(z, edge_attr, wc, we2, be2):
    eb = 2000
    grid = N_EDGES // eb
    return pl.pallas_call(
        _edge_mlp_body,
        grid=(grid,),
        in_specs=[
            pl.BlockSpec((eb, D), lambda i: (i, 0)),
            pl.BlockSpec((eb, 4), lambda i: (i, 0)),
            pl.BlockSpec((4, D), lambda i: (0, 0)),
            pl.BlockSpec((D, D), lambda i: (0, 0)),
            pl.BlockSpec((1, D), lambda i: (0, 0)),
        ],
        out_specs=pl.BlockSpec((eb, D), lambda i: (i, 0)),
        out_shape=jax.ShapeDtypeStruct((N_EDGES, D), jnp.float32),
    )(z, edge_attr, wc, we2, be2.reshape(1, D))


def _scatter_body(e2_hbm, dst2_hbm, zero_hbm, out_hbm, idst, rows, acc, sem, semsc):
    cid = lax.axis_index("c")
    sid = lax.axis_index("s")
    wid = sid * _NC + cid
    c0 = wid * _CPW
    row0 = sid * _ROWS_PER_TILE

    pltpu.sync_copy(
        zero_hbm.at[pl.ds(row0, _ROWS_PER_TILE)],
        acc.at[pl.ds(row0, _ROWS_PER_TILE)],
    )
    pltpu.sync_copy(dst2_hbm.at[pl.ds(c0, _CPW)], idst)
    plsc.subcore_barrier()

    def chunk(i, carry):
        base = (c0 + i) * _CHUNK

        @pl.when(base < N_EDGES)
        def _():
            pltpu.async_copy(e2_hbm.at[pl.ds(base, _CHUNK)], rows, sem).wait()
            pltpu.sync_copy(rows, acc.at[idst.at[i]], add=True)

        return carry

    lax.fori_loop(0, _CPW, chunk, 0)
    plsc.subcore_barrier()
    pltpu.sync_copy(
        acc.at[pl.ds(row0, _ROWS_PER_TILE)],
        out_hbm.at[cid, pl.ds(row0, _ROWS_PER_TILE)],
    )


_scatter = functools.partial(
    pl.kernel,
    out_type=jax.ShapeDtypeStruct((_NC, _N_PAD, D), jnp.float32),
    mesh=_sc_mesh,
    scratch_types=[
        pltpu.VMEM((_CPW, _CHUNK), jnp.int32),
        pltpu.VMEM((_CHUNK, D), jnp.float32),
        pltpu.VMEM_SHARED((_N_PAD, D), jnp.float32),
        pltpu.SemaphoreType.DMA,
        pltpu.SemaphoreType.DMA,
    ],
)(_scatter_body)


def _node_mlp_body(
    x_ref, p0_ref, p1_ref, wn1a_ref, wn1b_ref, bn1_ref, wn2_ref, bn2_ref, out_ref
):
    x = x_ref[...]
    msg = p0_ref[...] + p1_ref[...]
    h = (
        jnp.dot(x, wn1a_ref[...], preferred_element_type=jnp.float32)
        + jnp.dot(msg, wn1b_ref[...], preferred_element_type=jnp.float32)
        + bn1_ref[...]
    )
    h = h * jax.nn.sigmoid(h)
    out_ref[...] = (
        x + jnp.dot(h, wn2_ref[...], preferred_element_type=jnp.float32) + bn2_ref[...]
    )


def _node_mlp(x, partials, wn1a, wn1b, bn1, wn2, bn2):
    nb = 2000
    grid = N_NODES // nb
    return pl.pallas_call(
        _node_mlp_body,
        grid=(grid,),
        in_specs=[
            pl.BlockSpec((nb, D), lambda i: (i, 0)),
            pl.BlockSpec((nb, D), lambda i: (i, 0)),
            pl.BlockSpec((nb, D), lambda i: (i, 0)),
            pl.BlockSpec((D, D), lambda i: (0, 0)),
            pl.BlockSpec((D, D), lambda i: (0, 0)),
            pl.BlockSpec((1, D), lambda i: (0, 0)),
            pl.BlockSpec((D, D), lambda i: (0, 0)),
            pl.BlockSpec((1, D), lambda i: (0, 0)),
        ],
        out_specs=pl.BlockSpec((nb, D), lambda i: (i, 0)),
        out_shape=jax.ShapeDtypeStruct((N_NODES, D), jnp.float32),
    )(x, partials[0], partials[1], wn1a, wn1b, bn1.reshape(1, D), wn2, bn2.reshape(1, D))


@jax.jit
def kernel(x, edge_index, edge_attr, We1, be1, We2, be2, Wn1, bn1, Wn2, bn2):
    src = edge_index[0].astype(jnp.int32)
    dst = edge_index[1].astype(jnp.int32)
    pad = _E_PAD - N_EDGES
    src2 = jnp.pad(src, (0, pad)).reshape(_E_PAD // _CHUNK, _CHUNK)
    dst2 = jnp.pad(dst, (0, pad)).reshape(_E_PAD // _CHUNK, _CHUNK)
    wa = We1[:D]
    wb = We1[D : 2 * D]
    wc = We1[2 * D :]
    wn1a = Wn1[:D]
    wn1b = Wn1[D:]
    zeros = jnp.zeros((_N_PAD, D), jnp.float32)

    p, q = _node_proj(x, wa, wb, be1)
    z = _gather(p, q, src2, dst2)
    e2 = _edge_mlp(z, edge_attr, wc, We2, be2)
    partials = _scatter(e2, dst2, zeros)
    partials = partials[:, :N_NODES]
    return _node_mlp(x, partials, wn1a, wn1b, bn1, Wn2, bn2)


# speedup vs baseline: 4.7499x; 1.0742x over previous
import functools

import jax
import jax.numpy as jnp
from jax import lax
from jax.experimental import pallas as pl
from jax.experimental.pallas import tpu as pltpu
from jax.experimental.pallas import tpu_sc as plsc

N_NODES = 10000
N_EDGES = 320000
D = 128

_NC = 2
_NS = 16
_NW = _NC * _NS
_CHUNK = 128
_NCH = N_EDGES // _CHUNK
_K = 5
_CPW = 80
_E_PAD = _NW * _CPW * _CHUNK
_N_PAD = 10240
_ROWS_PER_TILE = _N_PAD // _NS

_sc_mesh = plsc.VectorSubcoreMesh(core_axis_name="c", subcore_axis_name="s")


def _pq_body(x_ref, wa_ref, wb_ref, be1_ref, p_ref, q_ref):
    x = x_ref[...]
    half_b = 0.5 * be1_ref[...]
    p_ref[...] = jnp.dot(x, wa_ref[...], preferred_element_type=jnp.float32) + half_b
    q_ref[...] = jnp.dot(x, wb_ref[...], preferred_element_type=jnp.float32) + half_b


def _node_proj(x, wa, wb, be1):
    nb = 2000
    grid = N_NODES // nb
    return pl.pallas_call(
        _pq_body,
        grid=(grid,),
        in_specs=[
            pl.BlockSpec((nb, D), lambda i: (i, 0)),
            pl.BlockSpec((D, D), lambda i: (0, 0)),
            pl.BlockSpec((D, D), lambda i: (0, 0)),
            pl.BlockSpec((1, D), lambda i: (0, 0)),
        ],
        out_specs=[
            pl.BlockSpec((nb, D), lambda i: (i, 0)),
            pl.BlockSpec((nb, D), lambda i: (i, 0)),
        ],
        out_shape=[
            jax.ShapeDtypeStruct((N_NODES, D), jnp.float32),
            jax.ShapeDtypeStruct((N_NODES, D), jnp.float32),
        ],
    )(x, wa, wb, be1.reshape(1, D))


def _gather_body(p_hbm, q_hbm, src2_hbm, dst2_hbm, z_hbm, isrc, idst, rows, sem, semw):
    wid = lax.axis_index("s") * _NC + lax.axis_index("c")
    c0 = wid * _CPW
    pltpu.sync_copy(src2_hbm.at[pl.ds(c0, _CPW)], isrc)
    pltpu.sync_copy(dst2_hbm.at[pl.ds(c0, _CPW)], idst)

    def group(g, carry):
        def ok(b):
            return (c0 + g * _K + b) * _CHUNK < N_EDGES

        for b in range(_K):
            @pl.when(ok(b))
            def _(b=b):
                pltpu.async_copy(p_hbm.at[isrc.at[g * _K + b]], rows.at[b], sem)
        for b in range(_K):
            @pl.when(ok(b))
            def _(b=b):
                pltpu.make_async_copy(
                    p_hbm.at[isrc.at[g * _K + b]], rows.at[b], sem
                ).wait()
        for b in range(_K):
            @pl.when(ok(b))
            def _(b=b):
                pltpu.async_copy(
                    q_hbm.at[idst.at[g * _K + b]], rows.at[b], sem, add=True
                )
        for b in range(_K):
            @pl.when(ok(b))
            def _(b=b):
                pltpu.make_async_copy(
                    q_hbm.at[idst.at[g * _K + b]], rows.at[b], sem
                ).wait()
        for b in range(_K):
            @pl.when(ok(b))
            def _(b=b):
                base = (c0 + g * _K + b) * _CHUNK
                pltpu.async_copy(rows.at[b], z_hbm.at[pl.ds(base, _CHUNK)], semw)
        for b in range(_K):
            @pl.when(ok(b))
            def _(b=b):
                base = (c0 + g * _K + b) * _CHUNK
                pltpu.make_async_copy(
                    rows.at[b], z_hbm.at[pl.ds(base, _CHUNK)], semw
                ).wait()
        return carry

    lax.fori_loop(0, _CPW // _K, group, 0)


_gather = functools.partial(
    pl.kernel,
    out_type=jax.ShapeDtypeStruct((N_EDGES, D), jnp.float32),
    mesh=_sc_mesh,
    scratch_types=[
        pltpu.VMEM((_CPW, _CHUNK), jnp.int32),
        pltpu.VMEM((_CPW, _CHUNK), jnp.int32),
        pltpu.VMEM((_K, _CHUNK, D), jnp.float32),
        pltpu.SemaphoreType.DMA,
        pltpu.SemaphoreType.DMA,
    ],
)(_gather_body)


def _edge_mlp_body(z_ref, ea_ref, wc_ref, we2_ref, be2_ref, out_ref):
    z = z_ref[...] + jnp.dot(
        ea_ref[...], wc_ref[...], preferred_element_type=jnp.float32
    )
    h = z * jax.nn.sigmoid(z)
    out_ref[...] = (
        jnp.dot(h, we2_ref[...], preferred_element_type=jnp.float32) + be2_ref[...]
    )


def _edge_mlp(z, edge_attr, wc, we2, be2):
    eb = 2000
    grid = N_EDGES // eb
    return pl.pallas_call(
        _edge_mlp_body,
        grid=(grid,),
        in_specs=[
            pl.BlockSpec((eb, D), lambda i: (i, 0)),
            pl.BlockSpec((eb, 4), lambda i: (i, 0)),
            pl.BlockSpec((4, D), lambda i: (0, 0)),
            pl.BlockSpec((D, D), lambda i: (0, 0)),
            pl.BlockSpec((1, D), lambda i: (0, 0)),
        ],
        out_specs=pl.BlockSpec((eb, D), lambda i: (i, 0)),
        out_shape=jax.ShapeDtypeStruct((N_EDGES, D), jnp.float32),
    )(z, edge_attr, wc, we2, be2.reshape(1, D))


def _scatter_body(e2_hbm, dst2_hbm, zero_hbm, out_hbm, idst, r0, r1, acc, sem, semsc):
    cid = lax.axis_index("c")
    sid = lax.axis_index("s")
    wid = sid * _NC + cid
    c0 = wid * _CPW
    row0 = sid * _ROWS_PER_TILE

    pltpu.sync_copy(
        zero_hbm.at[pl.ds(row0, _ROWS_PER_TILE)],
        acc.at[pl.ds(row0, _ROWS_PER_TILE)],
    )
    pltpu.sync_copy(dst2_hbm.at[pl.ds(c0, _CPW)], idst)
    plsc.subcore_barrier()

    def load(i, buf):
        @pl.when((c0 + i) * _CHUNK < N_EDGES)
        def _():
            pltpu.async_copy(e2_hbm.at[pl.ds((c0 + i) * _CHUNK, _CHUNK)], buf, sem)

    def drain(i, buf):
        @pl.when((c0 + i) * _CHUNK < N_EDGES)
        def _():
            pltpu.make_async_copy(
                e2_hbm.at[pl.ds((c0 + i) * _CHUNK, _CHUNK)], buf, sem
            ).wait()

    def scatter(i, buf):
        @pl.when((c0 + i) * _CHUNK < N_EDGES)
        def _():
            pltpu.sync_copy(buf, acc.at[idst.at[i]], add=True)

    load(0, r0)

    def pair(g, carry):
        i = g * 2
        drain(i, r0)
        load(i + 1, r1)
        scatter(i, r0)
        drain(i + 1, r1)
        load(i + 2, r0)
        scatter(i + 1, r1)
        return carry

    lax.fori_loop(0, _CPW // 2 - 1, pair, 0)
    i_last = _CPW - 2
    drain(i_last, r0)
    load(i_last + 1, r1)
    scatter(i_last, r0)
    drain(i_last + 1, r1)
    scatter(i_last + 1, r1)
    plsc.subcore_barrier()
    pltpu.sync_copy(
        acc.at[pl.ds(row0, _ROWS_PER_TILE)],
        out_hbm.at[cid, pl.ds(row0, _ROWS_PER_TILE)],
    )


_scatter = functools.partial(
    pl.kernel,
    out_type=jax.ShapeDtypeStruct((_NC, _N_PAD, D), jnp.float32),
    mesh=_sc_mesh,
    scratch_types=[
        pltpu.VMEM((_CPW, _CHUNK), jnp.int32),
        pltpu.VMEM((_CHUNK, D), jnp.float32),
        pltpu.VMEM((_CHUNK, D), jnp.float32),
        pltpu.VMEM_SHARED((_N_PAD, D), jnp.float32),
        pltpu.SemaphoreType.DMA,
        pltpu.SemaphoreType.DMA,
    ],
)(_scatter_body)


def _node_mlp_body(
    x_ref, p0_ref, p1_ref, wn1a_ref, wn1b_ref, bn1_ref, wn2_ref, bn2_ref, out_ref
):
    x = x_ref[...]
    msg = p0_ref[...] + p1_ref[...]
    h = (
        jnp.dot(x, wn1a_ref[...], preferred_element_type=jnp.float32)
        + jnp.dot(msg, wn1b_ref[...], preferred_element_type=jnp.float32)
        + bn1_ref[...]
    )
    h = h * jax.nn.sigmoid(h)
    out_ref[...] = (
        x + jnp.dot(h, wn2_ref[...], preferred_element_type=jnp.float32) + bn2_ref[...]
    )


def _node_mlp(x, partials, wn1a, wn1b, bn1, wn2, bn2):
    nb = 2000
    grid = N_NODES // nb
    return pl.pallas_call(
        _node_mlp_body,
        grid=(grid,),
        in_specs=[
            pl.BlockSpec((nb, D), lambda i: (i, 0)),
            pl.BlockSpec((nb, D), lambda i: (i, 0)),
            pl.BlockSpec((nb, D), lambda i: (i, 0)),
            pl.BlockSpec((D, D), lambda i: (0, 0)),
            pl.BlockSpec((D, D), lambda i: (0, 0)),
            pl.BlockSpec((1, D), lambda i: (0, 0)),
            pl.BlockSpec((D, D), lambda i: (0, 0)),
            pl.BlockSpec((1, D), lambda i: (0, 0)),
        ],
        out_specs=pl.BlockSpec((nb, D), lambda i: (i, 0)),
        out_shape=jax.ShapeDtypeStruct((N_NODES, D), jnp.float32),
    )(x, partials[0], partials[1], wn1a, wn1b, bn1.reshape(1, D), wn2, bn2.reshape(1, D))


@jax.jit
def kernel(x, edge_index, edge_attr, We1, be1, We2, be2, Wn1, bn1, Wn2, bn2):
    src = edge_index[0].astype(jnp.int32)
    dst = edge_index[1].astype(jnp.int32)
    pad = _E_PAD - N_EDGES
    src2 = jnp.pad(src, (0, pad)).reshape(_E_PAD // _CHUNK, _CHUNK)
    dst2 = jnp.pad(dst, (0, pad)).reshape(_E_PAD // _CHUNK, _CHUNK)
    wa = We1[:D]
    wb = We1[D : 2 * D]
    wc = We1[2 * D :]
    wn1a = Wn1[:D]
    wn1b = Wn1[D:]
    zeros = jnp.zeros((_N_PAD, D), jnp.float32)

    p, q = _node_proj(x, wa, wb, be1)
    z = _gather(p, q, src2, dst2)
    e2 = _edge_mlp(z, edge_attr, wc, We2, be2)
    partials = _scatter(e2, dst2, zeros)
    partials = partials[:, :N_NODES]
    return _node_mlp(x, partials, wn1a, wn1b, bn1, Wn2, bn2)


# speedup vs baseline: 5.1377x; 1.0817x over previous
import functools

import jax
import jax.numpy as jnp
from jax import lax
from jax.experimental import pallas as pl
from jax.experimental.pallas import tpu as pltpu
from jax.experimental.pallas import tpu_sc as plsc

N_NODES = 10000
N_EDGES = 320000
D = 128

_NC = 2
_NS = 16
_NW = _NC * _NS
_CHUNK = 128
_K = 5
_EH = N_EDGES // 2
_CPW = 40
_EH_PAD = _NW * _CPW * _CHUNK
_N_PAD = 10240
_ROWS_PER_TILE = _N_PAD // _NS

_sc_mesh = plsc.VectorSubcoreMesh(core_axis_name="c", subcore_axis_name="s")


def _pq_body(x_ref, wa_ref, wb_ref, be1_ref, p_ref, q_ref):
    x = x_ref[...]
    half_b = 0.5 * be1_ref[...]
    p_ref[...] = jnp.dot(x, wa_ref[...], preferred_element_type=jnp.float32) + half_b
    q_ref[...] = jnp.dot(x, wb_ref[...], preferred_element_type=jnp.float32) + half_b


def _node_proj(x, wa, wb, be1):
    nb = 2000
    grid = N_NODES // nb
    return pl.pallas_call(
        _pq_body,
        grid=(grid,),
        in_specs=[
            pl.BlockSpec((nb, D), lambda i: (i, 0)),
            pl.BlockSpec((D, D), lambda i: (0, 0)),
            pl.BlockSpec((D, D), lambda i: (0, 0)),
            pl.BlockSpec((1, D), lambda i: (0, 0)),
        ],
        out_specs=[
            pl.BlockSpec((nb, D), lambda i: (i, 0)),
            pl.BlockSpec((nb, D), lambda i: (i, 0)),
        ],
        out_shape=[
            jax.ShapeDtypeStruct((N_NODES, D), jnp.float32),
            jax.ShapeDtypeStruct((N_NODES, D), jnp.float32),
        ],
    )(x, wa, wb, be1.reshape(1, D))


def _gather_body(p_hbm, q_hbm, src2_hbm, dst2_hbm, z_hbm, isrc, idst, rows, sem, semw):
    wid = lax.axis_index("s") * _NC + lax.axis_index("c")
    c0 = wid * _CPW
    pltpu.sync_copy(src2_hbm.at[pl.ds(c0, _CPW)], isrc)
    pltpu.sync_copy(dst2_hbm.at[pl.ds(c0, _CPW)], idst)

    def group(g, carry):
        def ok(b):
            return (c0 + g * _K + b) * _CHUNK < _EH

        for b in range(_K):
            @pl.when(ok(b))
            def _(b=b):
                pltpu.async_copy(p_hbm.at[isrc.at[g * _K + b]], rows.at[b], sem)
        for b in range(_K):
            @pl.when(ok(b))
            def _(b=b):
                pltpu.make_async_copy(
                    p_hbm.at[isrc.at[g * _K + b]], rows.at[b], sem
                ).wait()
        for b in range(_K):
            @pl.when(ok(b))
            def _(b=b):
                pltpu.async_copy(
                    q_hbm.at[idst.at[g * _K + b]], rows.at[b], sem, add=True
                )
        for b in range(_K):
            @pl.when(ok(b))
            def _(b=b):
                pltpu.make_async_copy(
                    q_hbm.at[idst.at[g * _K + b]], rows.at[b], sem
                ).wait()
        for b in range(_K):
            @pl.when(ok(b))
            def _(b=b):
                base = (c0 + g * _K + b) * _CHUNK
                pltpu.async_copy(rows.at[b], z_hbm.at[pl.ds(base, _CHUNK)], semw)
        for b in range(_K):
            @pl.when(ok(b))
            def _(b=b):
                base = (c0 + g * _K + b) * _CHUNK
                pltpu.make_async_copy(
                    rows.at[b], z_hbm.at[pl.ds(base, _CHUNK)], semw
                ).wait()
        return carry

    lax.fori_loop(0, _CPW // _K, group, 0)


_gather = functools.partial(
    pl.kernel,
    out_type=jax.ShapeDtypeStruct((_EH, D), jnp.float32),
    mesh=_sc_mesh,
    scratch_types=[
        pltpu.VMEM((_CPW, _CHUNK), jnp.int32),
        pltpu.VMEM((_CPW, _CHUNK), jnp.int32),
        pltpu.VMEM((_K, _CHUNK, D), jnp.float32),
        pltpu.SemaphoreType.DMA,
        pltpu.SemaphoreType.DMA,
    ],
)(_gather_body)


def _edge_mlp_body(z_ref, ea_ref, wc_ref, we2_ref, be2_ref, out_ref):
    z = z_ref[...] + jnp.dot(
        ea_ref[...], wc_ref[...], preferred_element_type=jnp.float32
    )
    h = z * jax.nn.sigmoid(z)
    out_ref[...] = (
        jnp.dot(h, we2_ref[...], preferred_element_type=jnp.float32) + be2_ref[...]
    )


def _edge_mlp(z, edge_attr, wc, we2, be2):
    eb = 2000
    grid = _EH // eb
    return pl.pallas_call(
        _edge_mlp_body,
        grid=(grid,),
        in_specs=[
            pl.BlockSpec((eb, D), lambda i: (i, 0)),
            pl.BlockSpec((eb, 4), lambda i: (i, 0)),
            pl.BlockSpec((4, D), lambda i: (0, 0)),
            pl.BlockSpec((D, D), lambda i: (0, 0)),
            pl.BlockSpec((1, D), lambda i: (0, 0)),
        ],
        out_specs=pl.BlockSpec((eb, D), lambda i: (i, 0)),
        out_shape=jax.ShapeDtypeStruct((_EH, D), jnp.float32),
    )(z, edge_attr, wc, we2, be2.reshape(1, D))


def _scatter_body(e2_hbm, dst2_hbm, zero_hbm, out_hbm, idst, r0, r1, acc, sem, semsc):
    cid = lax.axis_index("c")
    sid = lax.axis_index("s")
    wid = sid * _NC + cid
    c0 = wid * _CPW
    row0 = sid * _ROWS_PER_TILE

    pltpu.sync_copy(
        zero_hbm.at[pl.ds(row0, _ROWS_PER_TILE)],
        acc.at[pl.ds(row0, _ROWS_PER_TILE)],
    )
    pltpu.sync_copy(dst2_hbm.at[pl.ds(c0, _CPW)], idst)
    plsc.subcore_barrier()

    def load(i, buf):
        @pl.when((c0 + i) * _CHUNK < _EH)
        def _():
            pltpu.async_copy(e2_hbm.at[pl.ds((c0 + i) * _CHUNK, _CHUNK)], buf, sem)

    def drain(i, buf):
        @pl.when((c0 + i) * _CHUNK < _EH)
        def _():
            pltpu.make_async_copy(
                e2_hbm.at[pl.ds((c0 + i) * _CHUNK, _CHUNK)], buf, sem
            ).wait()

    def scatter(i, buf):
        @pl.when((c0 + i) * _CHUNK < _EH)
        def _():
            pltpu.sync_copy(buf, acc.at[idst.at[i]], add=True)

    load(0, r0)

    def pair(g, carry):
        i = g * 2
        drain(i, r0)
        load(i + 1, r1)
        scatter(i, r0)
        drain(i + 1, r1)
        load(i + 2, r0)
        scatter(i + 1, r1)
        return carry

    lax.fori_loop(0, _CPW // 2 - 1, pair, 0)
    i_last = _CPW - 2
    drain(i_last, r0)
    load(i_last + 1, r1)
    scatter(i_last, r0)
    drain(i_last + 1, r1)
    scatter(i_last + 1, r1)

    plsc.subcore_barrier()
    pltpu.sync_copy(
        acc.at[pl.ds(row0, _ROWS_PER_TILE)],
        out_hbm.at[cid, pl.ds(row0, _ROWS_PER_TILE)],
    )


_scatter = functools.partial(
    pl.kernel,
    out_type=jax.ShapeDtypeStruct((_NC, _N_PAD, D), jnp.float32),
    mesh=_sc_mesh,
    scratch_types=[
        pltpu.VMEM((_CPW, _CHUNK), jnp.int32),
        pltpu.VMEM((_CHUNK, D), jnp.float32),
        pltpu.VMEM((_CHUNK, D), jnp.float32),
        pltpu.VMEM_SHARED((_N_PAD, D), jnp.float32),
        pltpu.SemaphoreType.DMA,
        pltpu.SemaphoreType.DMA,
    ],
)(_scatter_body)


def _node_mlp_body(
    x_ref, pa0_ref, pa1_ref, pb0_ref, pb1_ref,
    wn1a_ref, wn1b_ref, bn1_ref, wn2_ref, bn2_ref, out_ref
):
    x = x_ref[...]
    msg = (pa0_ref[...] + pa1_ref[...]) + (pb0_ref[...] + pb1_ref[...])
    h = (
        jnp.dot(x, wn1a_ref[...], preferred_element_type=jnp.float32)
        + jnp.dot(msg, wn1b_ref[...], preferred_element_type=jnp.float32)
        + bn1_ref[...]
    )
    h = h * jax.nn.sigmoid(h)
    out_ref[...] = (
        x + jnp.dot(h, wn2_ref[...], preferred_element_type=jnp.float32) + bn2_ref[...]
    )


def _node_mlp(x, pa, pb, wn1a, wn1b, bn1, wn2, bn2):
    nb = 2000
    grid = N_NODES // nb
    blk = pl.BlockSpec((nb, D), lambda i: (i, 0))
    wblk = pl.BlockSpec((D, D), lambda i: (0, 0))
    bblk = pl.BlockSpec((1, D), lambda i: (0, 0))
    return pl.pallas_call(
        _node_mlp_body,
        grid=(grid,),
        in_specs=[blk, blk, blk, blk, blk, wblk, wblk, bblk, wblk, bblk],
        out_specs=blk,
        out_shape=jax.ShapeDtypeStruct((N_NODES, D), jnp.float32),
    )(
        x, pa[0], pa[1], pb[0], pb[1],
        wn1a, wn1b, bn1.reshape(1, D), wn2, bn2.reshape(1, D),
    )


def _pack_idx(v):
    return jnp.pad(v, (0, _EH_PAD - _EH)).reshape(_EH_PAD // _CHUNK, _CHUNK)


@jax.jit
def kernel(x, edge_index, edge_attr, We1, be1, We2, be2, Wn1, bn1, Wn2, bn2):
    src = edge_index[0].astype(jnp.int32)
    dst = edge_index[1].astype(jnp.int32)
    wa = We1[:D]
    wb = We1[D : 2 * D]
    wc = We1[2 * D :]
    wn1a = Wn1[:D]
    wn1b = Wn1[D:]
    zeros = jnp.zeros((_N_PAD, D), jnp.float32)

    p, q = _node_proj(x, wa, wb, be1)
    parts = []
    for h in range(2):
        sl = slice(h * _EH, (h + 1) * _EH)
        src2 = _pack_idx(src[sl])
        dst2 = _pack_idx(dst[sl])
        z = _gather(p, q, src2, dst2)
        e2 = _edge_mlp(z, edge_attr[sl], wc, We2, be2)
        parts.append(_scatter(e2, dst2, zeros)[:, :N_NODES])
    return _node_mlp(x, parts[0], parts[1], wn1a, wn1b, bn1, Wn2, bn2)
